# Initial kernel scaffold; baseline (speedup 1.0000x reference)
import jax
import jax.numpy as jnp
from jax.experimental import pallas as pl


def kernel(x, edge_index, conv0_w, conv0_b, conv1_w, conv1_b, conv2_w, conv2_b, att0_w, att0_b, att1_w, att1_b, att2_w, att2_b, dr0_w, dr0_b, dr1_w, dr1_b, dr2_w, dr2_b, bn0_g, bn0_b, bn1_g, bn1_b):
    raise NotImplementedError("write your pallas kernel here")


# speedup vs baseline: 2.4163x; 2.4163x over previous
import functools

import jax
import jax.numpy as jnp
from jax import lax
from jax.experimental import pallas as pl
from jax.experimental.pallas import tpu as pltpu
from jax.experimental.pallas import tpu_sc as plsc

N = 10000
E = 320000
D = 128
K = 50
BN_EPS = 1e-5

NC = 2
NS = 16
NW = NC * NS
CH = 128
EPW = 10240
E_PAD = EPW * NW
N_PAD = 10112
ROWS_PER_TILE = N_PAD // NS

BLK = 1000


_sc_mesh = plsc.VectorSubcoreMesh(core_axis_name="c", subcore_axis_name="s")


@functools.partial(
    pl.kernel,
    out_type=jax.ShapeDtypeStruct((NC, N_PAD, D), jnp.float32),
    mesh=_sc_mesh,
    scratch_types=[
        pltpu.VMEM((CH,), jnp.int32),
        pltpu.VMEM((CH,), jnp.int32),
        pltpu.VMEM((CH, D), jnp.float32),
        pltpu.VMEM_SHARED((N_PAD, D), jnp.float32),
        pltpu.SemaphoreType.DMA,
    ],
)
def _seg_sum_sc(xw_hbm, src_hbm, dst_hbm, zeros_hbm, out_hbm,
                src_v, dst_v, rows_v, acc_sh, sem):
    cid = lax.axis_index("c")
    sid = lax.axis_index("s")
    wid = sid * NC + cid
    r0 = sid * ROWS_PER_TILE
    pltpu.sync_copy(zeros_hbm.at[pl.ds(r0, ROWS_PER_TILE)],
                    acc_sh.at[pl.ds(r0, ROWS_PER_TILE)])
    plsc.subcore_barrier()

    base = wid * EPW

    def body(i, carry):
        off = pl.multiple_of(base + i * CH, CH)
        pltpu.sync_copy(src_hbm.at[pl.ds(off, CH)], src_v)
        pltpu.sync_copy(dst_hbm.at[pl.ds(off, CH)], dst_v)
        pltpu.async_copy(xw_hbm.at[src_v], rows_v, sem).wait()
        pltpu.sync_copy(rows_v, acc_sh.at[dst_v], add=True)
        return carry

    lax.fori_loop(0, EPW // CH, body, 0)
    plsc.subcore_barrier()
    pltpu.sync_copy(acc_sh.at[pl.ds(r0, ROWS_PER_TILE)],
                    out_hbm.at[cid, pl.ds(r0, ROWS_PER_TILE)])


def _pre_body(h_ref, cw_ref, aw_ref, ab_ref,
              xw_ref, tmp_ref, vtz_ref, vsum_ref, usum_ref):
    pid = pl.program_id(0)
    h = h_ref[...]
    xw_ref[...] = jnp.dot(h, cw_ref[...], preferred_element_type=jnp.float32)
    tmp = jnp.maximum(
        jnp.dot(h, aw_ref[...], preferred_element_type=jnp.float32)
        + ab_ref[...], 0.0)
    tmp_ref[...] = tmp
    u = tmp[:, :K]
    v = tmp[:, K:2 * K]
    z = tmp[:, 2 * K:3 * K]
    vtz = lax.dot_general(v, z, (((0,), (0,)), ((), ())),
                          preferred_element_type=jnp.float32)
    vsum = jnp.sum(v, axis=0, keepdims=True)
    usum = jnp.sum(u, axis=0, keepdims=True)

    @pl.when(pid == 0)
    def _():
        vtz_ref[...] = vtz
        vsum_ref[...] = vsum
        usum_ref[...] = usum

    @pl.when(pid != 0)
    def _():
        vtz_ref[...] += vtz
        vsum_ref[...] += vsum
        usum_ref[...] += usum


def _pre_call(h, cw, aw, ab):
    grid = N // BLK
    return pl.pallas_call(
        _pre_body,
        grid=(grid,),
        in_specs=[
            pl.BlockSpec((BLK, D), lambda i: (i, 0)),
            pl.BlockSpec((D, D), lambda i: (0, 0)),
            pl.BlockSpec((D, 4 * K), lambda i: (0, 0)),
            pl.BlockSpec((1, 4 * K), lambda i: (0, 0)),
        ],
        out_specs=[
            pl.BlockSpec((BLK, D), lambda i: (i, 0)),
            pl.BlockSpec((BLK, 4 * K), lambda i: (i, 0)),
            pl.BlockSpec((K, K), lambda i: (0, 0)),
            pl.BlockSpec((1, K), lambda i: (0, 0)),
            pl.BlockSpec((1, K), lambda i: (0, 0)),
        ],
        out_shape=[
            jax.ShapeDtypeStruct((N, D), jnp.float32),
            jax.ShapeDtypeStruct((N, 4 * K), jnp.float32),
            jax.ShapeDtypeStruct((K, K), jnp.float32),
            jax.ShapeDtypeStruct((1, K), jnp.float32),
            jax.ShapeDtypeStruct((1, K), jnp.float32),
        ],
    )(h, cw, aw, ab)


def _post_body(tmp_ref, agg_ref, h_ref, cb_ref, vtz_ref, vsum_ref, usum_ref,
               w1_ref, w2_ref, w3_ref, w4_ref, b_ref, g_ref, beta_ref,
               out_ref, *, nrelu, bn):
    tmp = tmp_ref[...]
    u = tmp[:, :K]
    t = tmp[:, 3 * K:]
    agg = agg_ref[0] + agg_ref[1]
    x_local = jnp.maximum(agg + cb_ref[...], 0.0)
    res = jnp.dot(u, vtz_ref[...], preferred_element_type=jnp.float32)
    d = 1.0 / (jnp.sum(vsum_ref[...] * usum_ref[...]) / N + 1e-6)
    acc = (jnp.dot(res * d, w1_ref[...], preferred_element_type=jnp.float32)
           + jnp.dot(t, w2_ref[...], preferred_element_type=jnp.float32)
           + jnp.dot(x_local, w3_ref[...], preferred_element_type=jnp.float32)
           + jnp.dot(h_ref[...], w4_ref[...],
                     preferred_element_type=jnp.float32)
           + b_ref[...])
    for _ in range(nrelu):
        acc = jnp.maximum(acc, 0.0)
    if bn:
        acc = acc * g_ref[...] + beta_ref[...]
    out_ref[...] = acc


def _post_call(tmp, agg2, h, cb, vtz, vsum, usum, w1, w2, w3, w4, b,
               g_scaled, beta, nrelu, bn):
    grid = N // BLK
    body = functools.partial(_post_body, nrelu=nrelu, bn=bn)
    return pl.pallas_call(
        body,
        grid=(grid,),
        in_specs=[
            pl.BlockSpec((BLK, 4 * K), lambda i: (i, 0)),
            pl.BlockSpec((NC, BLK, D), lambda i: (0, i, 0)),
            pl.BlockSpec((BLK, D), lambda i: (i, 0)),
            pl.BlockSpec((1, D), lambda i: (0, 0)),
            pl.BlockSpec((K, K), lambda i: (0, 0)),
            pl.BlockSpec((1, K), lambda i: (0, 0)),
            pl.BlockSpec((1, K), lambda i: (0, 0)),
            pl.BlockSpec((K, D), lambda i: (0, 0)),
            pl.BlockSpec((K, D), lambda i: (0, 0)),
            pl.BlockSpec((D, D), lambda i: (0, 0)),
            pl.BlockSpec((D, D), lambda i: (0, 0)),
            pl.BlockSpec((1, D), lambda i: (0, 0)),
            pl.BlockSpec((1, D), lambda i: (0, 0)),
            pl.BlockSpec((1, D), lambda i: (0, 0)),
        ],
        out_specs=pl.BlockSpec((BLK, D), lambda i: (i, 0)),
        out_shape=jax.ShapeDtypeStruct((N, D), jnp.float32),
    )(tmp, agg2, h, cb, vtz, vsum, usum, w1, w2, w3, w4, b, g_scaled, beta)


def _layer(h, src_p, dst_p, zeros, conv_w, conv_b, att_w, att_b,
           dr_w, dr_b, g_scaled, beta, nrelu, bn):
    xw, tmp, vtz, vsum, usum = _pre_call(h, conv_w, att_w, att_b)
    agg2 = _seg_sum_sc(xw, src_p, dst_p, zeros)
    w1 = dr_w[:K]
    w2 = dr_w[K:2 * K]
    w3 = dr_w[2 * K:2 * K + D]
    w4 = dr_w[2 * K + D:]
    return _post_call(tmp, agg2, h, conv_b, vtz, vsum, usum,
                      w1, w2, w3, w4, dr_b, g_scaled, beta, nrelu, bn)


def kernel(x, edge_index, conv0_w, conv0_b, conv1_w, conv1_b, conv2_w,
           conv2_b, att0_w, att0_b, att1_w, att1_b, att2_w, att2_b,
           dr0_w, dr0_b, dr1_w, dr1_b, dr2_w, dr2_b,
           bn0_g, bn0_b, bn1_g, bn1_b):
    src = edge_index[0]
    dst = edge_index[1]
    pad = E_PAD - E
    src_p = jnp.concatenate([src, jnp.zeros((pad,), jnp.int32)])
    dst_p = jnp.concatenate([dst, jnp.full((pad,), N, jnp.int32)])
    zeros = jnp.zeros((N_PAD, D), jnp.float32)

    inv = 1.0 / jnp.sqrt(1.0 + BN_EPS)
    g0 = (bn0_g * inv).reshape(1, D)
    g1 = (bn1_g * inv).reshape(1, D)
    one = jnp.ones((1, D), jnp.float32)
    zero = jnp.zeros((1, D), jnp.float32)

    h = _layer(x, src_p, dst_p, zeros, conv0_w, conv0_b.reshape(1, D),
               att0_w, att0_b.reshape(1, 4 * K), dr0_w,
               dr0_b.reshape(1, D), g0, bn0_b.reshape(1, D), 2, True)
    h = _layer(h, src_p, dst_p, zeros, conv1_w, conv1_b.reshape(1, D),
               att1_w, att1_b.reshape(1, 4 * K), dr1_w,
               dr1_b.reshape(1, D), g1, bn1_b.reshape(1, D), 1, True)
    out = _layer(h, src_p, dst_p, zeros, conv2_w, conv2_b.reshape(1, D),
                 att2_w, att2_b.reshape(1, 4 * K), dr2_w,
                 dr2_b.reshape(1, D), one, zero, 0, False)
    return out
